# baseline (speedup 1.0000x reference)
import functools

import jax
import jax.numpy as jnp
from jax import lax
from jax.experimental import pallas as pl
from jax.experimental.pallas import tpu as pltpu
from jax.experimental.pallas import tpu_sc as plsc

_NUM_EXPERTS = 64
_TOPK = 2
_RANK = 8
_ALPHA = 16.0
_SCALE = _ALPHA / float(_RANK)
_LANES = 16
_N_WORKERS = 32


def _logits_body(h_ref, w_ref, lg_ref):
    lg_ref[...] = lax.dot_general(
        h_ref[...], w_ref[...], (((1,), (1,)), ((), ())),
        preferred_element_type=jnp.float32)


def _lows_body(h_ref, aq_ref, av_ref, lq_ref, lv_ref):
    f32 = jnp.float32
    bf = jnp.bfloat16
    hb = h_ref[...].astype(bf)
    lq_ref[...] = jnp.dot(hb, aq_ref[...].astype(bf),
                          preferred_element_type=f32).astype(bf)
    lv_ref[...] = jnp.dot(hb, av_ref[...].astype(bf),
                          preferred_element_type=f32).astype(bf)


def _router_sc_body(lg_hbm, gates_hbm, lg_v, gates_v):
    wid = lax.axis_index("s") * 2 + lax.axis_index("c")
    per = 2048 // _N_WORKERS
    base = wid * per
    pltpu.sync_copy(lg_hbm.at[pl.ds(base, per)], lg_v)
    neg = jnp.float32(-3.0e38)
    big = jnp.int32(10_000)

    def allmax(v):
        for sh in (8, 4, 2, 1):
            idx = (lax.iota(jnp.int32, _LANES) + sh) & (_LANES - 1)
            v = jnp.maximum(v, v.at[idx].get(mode="promise_in_bounds"))
        return v

    def allmin(v):
        for sh in (8, 4, 2, 1):
            idx = (lax.iota(jnp.int32, _LANES) + sh) & (_LANES - 1)
            v = jnp.minimum(v, v.at[idx].get(mode="promise_in_bounds"))
        return v

    def argtop(vs, cols, m):
        cands = [jnp.where(v == m, col, big) for v, col in zip(vs, cols)]
        return allmin(jnp.minimum(jnp.minimum(cands[0], cands[1]),
                                  jnp.minimum(cands[2], cands[3])))

    def token(t, carry):
        vs = [lg_v[t, pl.ds(c * _LANES, _LANES)] for c in range(4)]
        cols = [lax.iota(jnp.int32, _LANES) + c * _LANES for c in range(4)]
        m1 = allmax(jnp.maximum(jnp.maximum(vs[0], vs[1]),
                                jnp.maximum(vs[2], vs[3])))
        i1 = argtop(vs, cols, m1)
        v2s = [jnp.where(cols[c] == i1, neg, vs[c]) for c in range(4)]
        m2 = allmax(jnp.maximum(jnp.maximum(v2s[0], v2s[1]),
                                jnp.maximum(v2s[2], v2s[3])))
        i2 = argtop(v2s, cols, m2)
        ev = jnp.exp(m2 - m1)
        g1 = _SCALE / (1.0 + ev)
        g2 = g1 * ev
        zero = jnp.zeros((_LANES,), jnp.float32)
        for c in range(4):
            out_c = (jnp.where(cols[c] == i1, g1, zero)
                     + jnp.where(cols[c] == i2, g2, zero))
            gates_v[t, pl.ds(c * _LANES, _LANES)] = out_c
        return carry

    lax.fori_loop(0, per, token, 0)
    pltpu.sync_copy(gates_v, gates_hbm.at[pl.ds(base, per)])


def _stage_c_body(lq_ref, lv_ref, g_ref, bq_ref, bv_ref, qo_ref, vo_ref):
    f32 = jnp.float32
    bf = jnp.bfloat16
    gates = g_ref[...]
    erow = lax.broadcasted_iota(jnp.int32, (_NUM_EXPERTS, _NUM_EXPERTS * _RANK), 0)
    ecol = lax.broadcasted_iota(jnp.int32, (_NUM_EXPERTS, _NUM_EXPERTS * _RANK), 1)
    expand = jnp.where(erow == ecol // _RANK, 1.0, 0.0).astype(bf)
    gexp = jnp.dot(gates.astype(bf), expand, preferred_element_type=f32)
    glow_q = (lq_ref[...].astype(f32) * gexp).astype(bf)
    qo_ref[...] = jnp.dot(glow_q, bq_ref[...].astype(bf), preferred_element_type=f32)
    glow_v = (lv_ref[...].astype(f32) * gexp).astype(bf)
    vo_ref[...] = jnp.dot(glow_v, bv_ref[...].astype(bf), preferred_element_type=f32)


def kernel(hidden_states, router_weight, q_lora_a, q_lora_b, v_lora_a, v_lora_b):
    orig_shape = hidden_states.shape[:-1]
    d_model = hidden_states.shape[-1]
    h = hidden_states.reshape(-1, d_model)
    n = h.shape[0]
    e, _, r = q_lora_a.shape
    q_out = q_lora_b.shape[-1]
    v_out = v_lora_b.shape[-1]
    f32 = jnp.float32
    bf = jnp.bfloat16

    aq = q_lora_a.transpose(1, 0, 2).reshape(d_model, e * r)
    av = v_lora_a.transpose(1, 0, 2).reshape(d_model, e * r)
    bq = q_lora_b.reshape(e * r, q_out)
    bv = v_lora_b.reshape(e * r, v_out)

    tile = 1024
    grid = (n // tile,)
    const_spec = lambda shape: pl.BlockSpec(shape, lambda i: (0, 0))

    logits = pl.pallas_call(
        _logits_body,
        grid=grid,
        in_specs=[
            pl.BlockSpec((tile, d_model), lambda i: (i, 0)),
            const_spec((e, d_model)),
        ],
        out_specs=pl.BlockSpec((tile, e), lambda i: (i, 0)),
        out_shape=jax.ShapeDtypeStruct((n, e), f32),
    )(h, router_weight)

    low_q, low_v = pl.pallas_call(
        _lows_body,
        grid=grid,
        in_specs=[
            pl.BlockSpec((tile, d_model), lambda i: (i, 0)),
            const_spec((d_model, e * r)),
            const_spec((d_model, e * r)),
        ],
        out_specs=[
            pl.BlockSpec((tile, e * r), lambda i: (i, 0)),
            pl.BlockSpec((tile, e * r), lambda i: (i, 0)),
        ],
        out_shape=[
            jax.ShapeDtypeStruct((n, e * r), bf),
            jax.ShapeDtypeStruct((n, e * r), bf),
        ],
    )(h, aq, av)

    per = n // _N_WORKERS
    router = functools.partial(
        pl.kernel,
        mesh=plsc.VectorSubcoreMesh(core_axis_name="c", subcore_axis_name="s"),
        out_type=jax.ShapeDtypeStruct((n, e), f32),
        scratch_types=[
            pltpu.VMEM((per, e), f32),
            pltpu.VMEM((per, e), f32),
        ],
    )(_router_sc_body)
    gates = router(logits)

    qo, vo = pl.pallas_call(
        _stage_c_body,
        grid=grid,
        in_specs=[
            pl.BlockSpec((tile, e * r), lambda i: (i, 0)),
            pl.BlockSpec((tile, e * r), lambda i: (i, 0)),
            pl.BlockSpec((tile, e), lambda i: (i, 0)),
            const_spec((e * r, q_out)),
            const_spec((e * r, v_out)),
        ],
        out_specs=[
            pl.BlockSpec((tile, q_out), lambda i: (i, 0)),
            pl.BlockSpec((tile, v_out), lambda i: (i, 0)),
        ],
        out_shape=[
            jax.ShapeDtypeStruct((n, q_out), f32),
            jax.ShapeDtypeStruct((n, v_out), f32),
        ],
    )(low_q, low_v, gates, bq, bv)
    return (qo.reshape(orig_shape + (q_out,)), vo.reshape(orig_shape + (v_out,)))


# speedup vs baseline: 1.9249x; 1.9249x over previous
import jax
import jax.numpy as jnp
from jax.experimental import pallas as pl

_NUM_EXPERTS = 64
_TOPK = 2
_RANK = 8
_ALPHA = 16.0
_SCALE = _ALPHA / float(_RANK)


def _fused_body(h_ref, w_ref, aq_ref, av_ref, bq_ref, bv_ref, qo_ref, vo_ref):
    h = h_ref[...]
    f32 = jnp.float32
    bf = jnp.bfloat16
    logits = jax.lax.dot_general(
        h, w_ref[...], (((1,), (1,)), ((), ())), preferred_element_type=f32)
    col = jax.lax.broadcasted_iota(jnp.int32, logits.shape, 1)
    big = jnp.int32(2 ** 30)
    m1 = jnp.max(logits, axis=-1, keepdims=True)
    i1 = jnp.min(jnp.where(logits == m1, col, big), axis=-1, keepdims=True)
    masked = jnp.where(col == i1, jnp.finfo(f32).min, logits)
    m2 = jnp.max(masked, axis=-1, keepdims=True)
    i2 = jnp.min(jnp.where(masked == m2, col, big), axis=-1, keepdims=True)
    e21 = jnp.exp(m2 - m1)
    denom = 1.0 + e21
    g1 = _SCALE / denom
    g2 = _SCALE * e21 / denom
    t = h.shape[0]
    ecol = jax.lax.broadcasted_iota(jnp.int32, (t, _NUM_EXPERTS * _RANK), 1) // _RANK
    gexp = jnp.where(ecol == i1, g1, 0.0) + jnp.where(ecol == i2, g2, 0.0)

    hb = h.astype(bf)
    low_q = jnp.dot(hb, aq_ref[...].astype(bf), preferred_element_type=f32)
    glow_q = (low_q * gexp).astype(bf)
    qo_ref[...] = jnp.dot(glow_q, bq_ref[...].astype(bf), preferred_element_type=f32)
    low_v = jnp.dot(hb, av_ref[...].astype(bf), preferred_element_type=f32)
    glow_v = (low_v * gexp).astype(bf)
    vo_ref[...] = jnp.dot(glow_v, bv_ref[...].astype(bf), preferred_element_type=f32)


def kernel(hidden_states, router_weight, q_lora_a, q_lora_b, v_lora_a, v_lora_b):
    orig_shape = hidden_states.shape[:-1]
    d_model = hidden_states.shape[-1]
    h = hidden_states.reshape(-1, d_model)
    n = h.shape[0]
    e, _, r = q_lora_a.shape
    q_out = q_lora_b.shape[-1]
    v_out = v_lora_b.shape[-1]

    aq = q_lora_a.transpose(1, 0, 2).reshape(d_model, e * r)
    av = v_lora_a.transpose(1, 0, 2).reshape(d_model, e * r)
    bq = q_lora_b.reshape(e * r, q_out)
    bv = v_lora_b.reshape(e * r, v_out)

    tile = 1024
    grid = (n // tile,)
    const_spec = lambda shape: pl.BlockSpec(shape, lambda i: (0, 0))
    qo, vo = pl.pallas_call(
        _fused_body,
        grid=grid,
        in_specs=[
            pl.BlockSpec((tile, d_model), lambda i: (i, 0)),
            const_spec((e, d_model)),
            const_spec((d_model, e * r)),
            const_spec((d_model, e * r)),
            const_spec((e * r, q_out)),
            const_spec((e * r, v_out)),
        ],
        out_specs=[
            pl.BlockSpec((tile, q_out), lambda i: (i, 0)),
            pl.BlockSpec((tile, v_out), lambda i: (i, 0)),
        ],
        out_shape=[
            jax.ShapeDtypeStruct((n, q_out), jnp.float32),
            jax.ShapeDtypeStruct((n, v_out), jnp.float32),
        ],
    )(h, router_weight, aq, av, bq, bv)
    return (qo.reshape(orig_shape + (q_out,)), vo.reshape(orig_shape + (v_out,)))
